# Initial kernel scaffold; baseline (speedup 1.0000x reference)
import jax
import jax.numpy as jnp
from jax.experimental import pallas as pl


def kernel(x, W0, as0, ad0, b0, W1, as1, ad1, b1, W2, as2, ad2, b2, W3, as3, ad3, b3, g0, be0, g1, be1, g2, be2, g3, be3, fc1_w, fc1_b, fc2_w, fc2_b, edge_index, batch):
    raise NotImplementedError("write your pallas kernel here")


# speedup vs baseline: 32.7667x; 32.7667x over previous
import functools
import jax
import jax.numpy as jnp
from jax import lax
from jax.experimental import pallas as pl
from jax.experimental.pallas import tpu as pltpu
from jax.experimental.pallas import tpu_sc as plsc

N = 10000
E = 160000
D = 128
H = 8
OC = 16
G = 128

NP = 10048
NW = 32
EPT = 5120
EP = NW * EPT
C = 128
NCH = EPT // C
DUMMY = N
STRIPE = NP // 16

BN = 1256
NB = NP // BN


def _sc_edge_body(h_hbm, as_hbm, ad_hbm, src_hbm, dst_hbm,
                  msg_out, den_out,
                  idx_s, idx_d, asr, adr, hr, msg, exb,
                  acc_m, acc_d, sem1, sem2, sem3):
    cid = lax.axis_index("c")
    sid = lax.axis_index("s")
    wid = cid * 16 + sid

    zero = jnp.zeros((16,), jnp.float32)

    def zrow(i, _):
        for j in range(D // 16):
            msg[i, pl.ds(j * 16, 16)] = zero
        exb[i, :] = zero
        return 0
    lax.fori_loop(0, C, zrow, 0)
    for k in range(STRIPE // C + 1):
        r0 = sid * STRIPE + k * C
        nr = min(C, STRIPE - k * C)
        if nr <= 0:
            break
        pltpu.sync_copy(msg.at[pl.ds(0, nr)], acc_m.at[pl.ds(r0, nr)])
        pltpu.sync_copy(exb.at[pl.ds(0, nr)], acc_d.at[pl.ds(r0, nr)])
    plsc.subcore_barrier()

    lanes = lax.iota(jnp.int32, 16)
    headmask = lanes < 8

    ebase = wid * EPT

    def chunk(k, _):
        base = ebase + k * C
        pltpu.sync_copy(src_hbm.at[pl.ds(base, C)], idx_s)
        pltpu.sync_copy(dst_hbm.at[pl.ds(base, C)], idx_d)
        c1 = pltpu.async_copy(h_hbm.at[idx_s], hr, sem1)
        c2 = pltpu.async_copy(as_hbm.at[idx_s], asr, sem2)
        c3 = pltpu.async_copy(ad_hbm.at[idx_d], adr, sem3)
        c2.wait()
        c3.wait()
        c1.wait()

        def edge(c, _):
            ev = asr[c, :] + adr[c, :]
            ev = jnp.where(ev > 0, ev, 0.2 * ev)
            ex = jnp.where(headmask, jnp.exp(ev), 0.0)
            exb[c, :] = ex
            for hh in range(H):
                msg[c, pl.ds(hh * 16, 16)] = hr[c, pl.ds(hh * 16, 16)] * ex[hh]
            return 0
        lax.fori_loop(0, C, edge, 0)

        pltpu.sync_copy(msg, acc_m.at[idx_d], add=True)
        pltpu.sync_copy(exb, acc_d.at[idx_d], add=True)
        return 0
    lax.fori_loop(0, NCH, chunk, 0)

    plsc.subcore_barrier()

    for k in range(STRIPE // C + 1):
        r0 = sid * STRIPE + k * C
        nr = min(C, STRIPE - k * C)
        if nr <= 0:
            break
        pltpu.sync_copy(acc_m.at[pl.ds(r0, nr)],
                        msg_out.at[cid, pl.ds(r0, nr)])
        pltpu.sync_copy(acc_d.at[pl.ds(r0, nr)],
                        den_out.at[cid, pl.ds(r0, nr)])


@functools.cache
def _get_sc_edge():
    return pl.kernel(
        _sc_edge_body,
        out_type=(
            jax.ShapeDtypeStruct((2, NP, D), jnp.float32),
            jax.ShapeDtypeStruct((2, NP, 16), jnp.float32),
        ),
        mesh=plsc.VectorSubcoreMesh(core_axis_name="c", subcore_axis_name="s"),
        compiler_params=pltpu.CompilerParams(use_tc_tiling_on_sc=False),
        scratch_types=[
        pltpu.VMEM((C,), jnp.int32),
        pltpu.VMEM((C,), jnp.int32),
        pltpu.VMEM((C, 16), jnp.float32),
        pltpu.VMEM((C, 16), jnp.float32),
        pltpu.VMEM((C, D), jnp.float32),
        pltpu.VMEM((C, D), jnp.float32),
        pltpu.VMEM((C, 16), jnp.float32),
        pltpu.VMEM_SHARED((NP, D), jnp.float32),
        pltpu.VMEM_SHARED((NP, 16), jnp.float32),
        pltpu.SemaphoreType.DMA,
        pltpu.SemaphoreType.DMA,
        pltpu.SemaphoreType.DMA,
        ],
    )


def _sc_edge(h, als, ald, srcp, dstp):
    return _get_sc_edge()(h, als, ald, srcp, dstp)


def _s16():
    j = lax.broadcasted_iota(jnp.int32, (D, 16), 0)
    i = lax.broadcasted_iota(jnp.int32, (D, 16), 1)
    return (i == j // OC).astype(jnp.float32)


def _rexp():
    i = lax.broadcasted_iota(jnp.int32, (16, D), 0)
    j = lax.broadcasted_iota(jnp.int32, (16, D), 1)
    return (i == j // OC).astype(jnp.float32)


def _mhead():
    j = lax.broadcasted_iota(jnp.int32, (D, 16), 0)
    i = lax.broadcasted_iota(jnp.int32, (D, 16), 1)
    return (i == j % OC).astype(jnp.float32) / H


def _elu(x):
    return jnp.where(x > 0, x, jnp.exp(x) - 1.0)


def _attn_logits(h, asv, adv):
    s16 = _s16()
    als = jnp.dot(h * asv, s16, preferred_element_type=jnp.float32)
    ald = jnp.dot(h * adv, s16, preferred_element_type=jnp.float32)
    return als, ald


def _p0_body(x_ref, w_ref, asv_ref, adv_ref, h_ref, as_ref, ad_ref):
    h = jnp.dot(x_ref[...], w_ref[...], preferred_element_type=jnp.float32)
    h_ref[...] = h
    als, ald = _attn_logits(h, asv_ref[...], adv_ref[...])
    as_ref[...] = als
    ad_ref[...] = ald


_p0 = pl.pallas_call(
    _p0_body,
    grid=(NB,),
    in_specs=[
        pl.BlockSpec((BN, D), lambda b: (b, 0)),
        pl.BlockSpec((D, D), lambda b: (0, 0)),
        pl.BlockSpec((1, D), lambda b: (0, 0)),
        pl.BlockSpec((1, D), lambda b: (0, 0)),
    ],
    out_specs=[
        pl.BlockSpec((BN, D), lambda b: (b, 0)),
        pl.BlockSpec((BN, 16), lambda b: (b, 0)),
        pl.BlockSpec((BN, 16), lambda b: (b, 0)),
    ],
    out_shape=[
        jax.ShapeDtypeStruct((NP, D), jnp.float32),
        jax.ShapeDtypeStruct((NP, 16), jnp.float32),
        jax.ShapeDtypeStruct((NP, 16), jnp.float32),
    ],
)


def _f_body(accm_ref, accd_ref, bias_ref, g_ref, be_ref,
            wn_ref, asv_ref, adv_ref,
            h_ref, as_ref, ad_ref,
            stash, s1, s2):
    p = pl.program_id(0)
    b = pl.program_id(1)

    @pl.when(p == 0)
    def _():
        @pl.when(b == 0)
        def _():
            s1[...] = jnp.zeros_like(s1)
            s2[...] = jnp.zeros_like(s2)
        a = accm_ref[0] + accm_ref[1]
        den = accd_ref[0] + accd_ref[1]
        den_rep = jnp.dot(den, _rexp(), preferred_element_type=jnp.float32)
        out = a / (den_rep + 1e-16) + bias_ref[...]
        rows = lax.broadcasted_iota(jnp.int32, (BN, 1), 0) + b * BN
        out = jnp.where(rows < N, out, 0.0)
        stash[pl.ds(b * BN, BN), :] = out
        s1[...] += jnp.sum(out, axis=0, keepdims=True)
        s2[...] += jnp.sum(out * out, axis=0, keepdims=True)

    @pl.when(p == 1)
    def _():
        mu = s1[...] / N
        var = s2[...] / N - mu * mu
        inv = g_ref[...] / jnp.sqrt(var + 1e-5)
        y = (stash[pl.ds(b * BN, BN), :] - mu) * inv + be_ref[...]
        y = _elu(y)
        hn = jnp.dot(y, wn_ref[...], preferred_element_type=jnp.float32)
        h_ref[...] = hn
        als, ald = _attn_logits(hn, asv_ref[...], adv_ref[...])
        as_ref[...] = als
        ad_ref[...] = ald


_f_layer = pl.pallas_call(
    _f_body,
    grid=(2, NB),
    in_specs=[
        pl.BlockSpec((2, BN, D), lambda p, b: (0, b, 0)),
        pl.BlockSpec((2, BN, 16), lambda p, b: (0, b, 0)),
        pl.BlockSpec((1, D), lambda p, b: (0, 0)),
        pl.BlockSpec((1, D), lambda p, b: (0, 0)),
        pl.BlockSpec((1, D), lambda p, b: (0, 0)),
        pl.BlockSpec((D, D), lambda p, b: (0, 0)),
        pl.BlockSpec((1, D), lambda p, b: (0, 0)),
        pl.BlockSpec((1, D), lambda p, b: (0, 0)),
    ],
    out_specs=[
        pl.BlockSpec((BN, D), lambda p, b: (b, 0)),
        pl.BlockSpec((BN, 16), lambda p, b: (b, 0)),
        pl.BlockSpec((BN, 16), lambda p, b: (b, 0)),
    ],
    out_shape=[
        jax.ShapeDtypeStruct((NP, D), jnp.float32),
        jax.ShapeDtypeStruct((NP, 16), jnp.float32),
        jax.ShapeDtypeStruct((NP, 16), jnp.float32),
    ],
    scratch_shapes=[
        pltpu.VMEM((NP, D), jnp.float32),
        pltpu.VMEM((1, D), jnp.float32),
        pltpu.VMEM((1, D), jnp.float32),
    ],
)


def _f3_body(accm_ref, accd_ref, bias_ref, g_ref, be_ref, batch_ref,
             fc1w_ref, fc1b_ref, fc2w_ref, fc2b_ref,
             out_ref,
             stash, s1, s2, pooled, cnt):
    p = pl.program_id(0)
    b = pl.program_id(1)

    @pl.when(p == 0)
    def _():
        @pl.when(b == 0)
        def _():
            s1[...] = jnp.zeros_like(s1)
            s2[...] = jnp.zeros_like(s2)
        a = accm_ref[0] + accm_ref[1]
        den = accd_ref[0] + accd_ref[1]
        den_rep = jnp.dot(den, _rexp(), preferred_element_type=jnp.float32)
        out128 = a / (den_rep + 1e-16)
        out16 = jnp.dot(out128, _mhead(), preferred_element_type=jnp.float32)
        out16 = out16 + bias_ref[...]
        rows = lax.broadcasted_iota(jnp.int32, (BN, 1), 0) + b * BN
        out16 = jnp.where(rows < N, out16, 0.0)
        stash[pl.ds(b * BN, BN), :] = out16
        s1[...] += jnp.sum(out16, axis=0, keepdims=True)
        s2[...] += jnp.sum(out16 * out16, axis=0, keepdims=True)

    @pl.when(p == 1)
    def _():
        @pl.when(b == 0)
        def _():
            pooled[...] = jnp.zeros_like(pooled)
            cnt[...] = jnp.zeros_like(cnt)
        mu = s1[...] / N
        var = s2[...] / N - mu * mu
        inv = g_ref[...] / jnp.sqrt(var + 1e-5)
        y = (stash[pl.ds(b * BN, BN), :] - mu) * inv + be_ref[...]
        y = _elu(y)
        seg = batch_ref[...]
        gcol = lax.broadcasted_iota(jnp.int32, (BN, G), 1)
        onehot = (seg == gcol).astype(jnp.float32)
        pooled[...] += lax.dot_general(
            onehot, y, (((0,), (0,)), ((), ())),
            preferred_element_type=jnp.float32)
        cnt[...] += jnp.sum(onehot, axis=0, keepdims=True)

        @pl.when(b == NB - 1)
        def _():
            cnts = jnp.maximum(cnt[...], 1.0)
            pm = pooled[...] / cnts.reshape(G, 1)
            o = _elu(jnp.dot(pm, fc1w_ref[...],
                             preferred_element_type=jnp.float32)
                     + fc1b_ref[...])
            out_ref[...] = (jnp.dot(o, fc2w_ref[...],
                                    preferred_element_type=jnp.float32)
                            + fc2b_ref[...])


_f3 = pl.pallas_call(
    _f3_body,
    grid=(2, NB),
    in_specs=[
        pl.BlockSpec((2, BN, D), lambda p, b: (0, b, 0)),
        pl.BlockSpec((2, BN, 16), lambda p, b: (0, b, 0)),
        pl.BlockSpec((1, 16), lambda p, b: (0, 0)),
        pl.BlockSpec((1, 16), lambda p, b: (0, 0)),
        pl.BlockSpec((1, 16), lambda p, b: (0, 0)),
        pl.BlockSpec((BN, 1), lambda p, b: (b, 0)),
        pl.BlockSpec((OC, D), lambda p, b: (0, 0)),
        pl.BlockSpec((1, D), lambda p, b: (0, 0)),
        pl.BlockSpec((D, D), lambda p, b: (0, 0)),
        pl.BlockSpec((1, D), lambda p, b: (0, 0)),
    ],
    out_specs=pl.BlockSpec((G, D), lambda p, b: (0, 0)),
    out_shape=jax.ShapeDtypeStruct((G, D), jnp.float32),
    scratch_shapes=[
        pltpu.VMEM((NP, 16), jnp.float32),
        pltpu.VMEM((1, 16), jnp.float32),
        pltpu.VMEM((1, 16), jnp.float32),
        pltpu.VMEM((G, 16), jnp.float32),
        pltpu.VMEM((1, G), jnp.float32),
    ],
)


def kernel(x, W0, as0, ad0, b0, W1, as1, ad1, b1, W2, as2, ad2, b2,
           W3, as3, ad3, b3, g0, be0, g1, be1, g2, be2, g3, be3,
           fc1_w, fc1_b, fc2_w, fc2_b, edge_index, batch):
    f32 = jnp.float32
    x_pad = jnp.zeros((NP, D), f32).at[:N].set(x.astype(f32))
    src = edge_index[0].astype(jnp.int32)
    dst = edge_index[1].astype(jnp.int32)
    pad = jnp.full((EP - E,), DUMMY, jnp.int32)
    srcp = jnp.concatenate([src, pad])
    dstp = jnp.concatenate([dst, pad])
    batch_pad = jnp.full((NP, 1), 999, jnp.int32).at[:N, 0].set(
        batch.astype(jnp.int32))

    Ws = [W0, W1, W2, W3]
    asvs = [a.reshape(1, D) for a in (as0, as1, as2, as3)]
    advs = [a.reshape(1, D) for a in (ad0, ad1, ad2, ad3)]
    biases = [b0.reshape(1, D), b1.reshape(1, D), b2.reshape(1, D)]
    gs = [g0.reshape(1, D), g1.reshape(1, D), g2.reshape(1, D)]
    bes = [be0.reshape(1, D), be1.reshape(1, D), be2.reshape(1, D)]

    h, als, ald = _p0(x_pad, Ws[0], asvs[0], advs[0])
    for i in range(3):
        accm, accd = _sc_edge(h, als, ald, srcp, dstp)
        h, als, ald = _f_layer(accm, accd, biases[i], gs[i], bes[i],
                               Ws[i + 1], asvs[i + 1], advs[i + 1])
    accm, accd = _sc_edge(h, als, ald, srcp, dstp)
    out = _f3(accm, accd, b3.reshape(1, OC), g3.reshape(1, OC),
              be3.reshape(1, OC), batch_pad,
              fc1_w, fc1_b.reshape(1, D), fc2_w, fc2_b.reshape(1, D))
    return out


# speedup vs baseline: 42.1014x; 1.2849x over previous
import functools
import jax
import jax.numpy as jnp
from jax import lax
from jax.experimental import pallas as pl
from jax.experimental.pallas import tpu as pltpu
from jax.experimental.pallas import tpu_sc as plsc

N = 10000
E = 160000
D = 128
H = 8
OC = 16
G = 128

NP = 10048
NW = 32
EPT = 5120
EP = NW * EPT
C = 64
NCH = EPT // C
DUMMY = N
STRIPE = NP // 16

BN = 1256
NB = NP // BN


def _sc_edge_body(h_hbm, as_hbm, ad_hbm, src_hbm, dst_hbm,
                  msg_out, den_out,
                  srcb, dstb, asr, adr, hr, msg, exb,
                  acc_m, acc_d, sem1, sem2, sem3, sem4):
    cid = lax.axis_index("c")
    sid = lax.axis_index("s")
    wid = cid * 16 + sid

    zero = jnp.zeros((16,), jnp.float32)

    def zrow(i, _):
        for j in range(D // 16):
            msg[i, pl.ds(j * 16, 16)] = zero
        exb[i, :] = zero
        return 0
    lax.fori_loop(0, C, zrow, 0)
    for k in range(STRIPE // C + 1):
        r0 = sid * STRIPE + k * C
        nr = min(C, STRIPE - k * C)
        if nr <= 0:
            break
        pltpu.sync_copy(msg.at[pl.ds(0, nr)], acc_m.at[pl.ds(r0, nr)])
        pltpu.sync_copy(exb.at[pl.ds(0, nr)], acc_d.at[pl.ds(r0, nr)])
    plsc.subcore_barrier()

    lanes = lax.iota(jnp.int32, 16)
    headmask = lanes < 8

    def fire_idx(k, p):
        pltpu.async_copy(src_hbm.at[wid, k], srcb.at[p], sem4)
        pltpu.async_copy(dst_hbm.at[wid, k], dstb.at[p], sem4)

    def drain_idx(p):
        pltpu.make_async_copy(src_hbm.at[wid, 0], srcb.at[p], sem4).wait()
        pltpu.make_async_copy(dst_hbm.at[wid, 0], dstb.at[p], sem4).wait()

    def fire_gather(p):
        pltpu.async_copy(h_hbm.at[srcb.at[p]], hr.at[p], sem1)
        pltpu.async_copy(as_hbm.at[srcb.at[p]], asr.at[p], sem2)
        pltpu.async_copy(ad_hbm.at[dstb.at[p]], adr.at[p], sem3)

    def drain_gather(p):
        pltpu.make_async_copy(h_hbm.at[srcb.at[p]], hr.at[p], sem1).wait()
        pltpu.make_async_copy(as_hbm.at[srcb.at[p]], asr.at[p], sem2).wait()
        pltpu.make_async_copy(ad_hbm.at[dstb.at[p]], adr.at[p], sem3).wait()

    def compute(p):
        def edge(c, _):
            ev = asr[p, c, :] + adr[p, c, :]
            ev = jnp.where(ev > 0, ev, 0.2 * ev)
            ex = jnp.where(headmask, jnp.exp(ev), 0.0)
            exb[c, :] = ex
            for hh in range(H):
                msg[c, pl.ds(hh * 16, 16)] = (
                    hr[p, c, pl.ds(hh * 16, 16)] * ex[hh])
            return 0
        lax.fori_loop(0, C, edge, 0, unroll=2)
        pltpu.sync_copy(msg, acc_m.at[dstb.at[p]], add=True)
        pltpu.sync_copy(exb, acc_d.at[dstb.at[p]], add=True)

    pltpu.sync_copy(src_hbm.at[wid, 0], srcb.at[0])
    pltpu.sync_copy(dst_hbm.at[wid, 0], dstb.at[0])
    fire_gather(0)
    fire_idx(1, 1)

    def two_chunks(j, _):
        k = 2 * j
        for p in (0, 1):
            drain_gather(p)
            drain_idx(1 - p)
            fire_gather(1 - p)
            compute(p)
            fire_idx(k + p + 2, p)
        return 0
    lax.fori_loop(0, NCH // 2, two_chunks, 0)
    drain_gather(0)
    drain_idx(1)

    plsc.subcore_barrier()

    for k in range(STRIPE // C + 1):
        r0 = sid * STRIPE + k * C
        nr = min(C, STRIPE - k * C)
        if nr <= 0:
            break
        pltpu.sync_copy(acc_m.at[pl.ds(r0, nr)],
                        msg_out.at[cid, pl.ds(r0, nr)])
        pltpu.sync_copy(acc_d.at[pl.ds(r0, nr)],
                        den_out.at[cid, pl.ds(r0, nr)])


@functools.cache
def _get_sc_edge():
    return pl.kernel(
        _sc_edge_body,
        out_type=(
            jax.ShapeDtypeStruct((2, NP, D), jnp.float32),
            jax.ShapeDtypeStruct((2, NP, 16), jnp.float32),
        ),
        mesh=plsc.VectorSubcoreMesh(core_axis_name="c", subcore_axis_name="s"),
        compiler_params=pltpu.CompilerParams(use_tc_tiling_on_sc=False),
        scratch_types=[
        pltpu.VMEM((2, C), jnp.int32),
        pltpu.VMEM((2, C), jnp.int32),
        pltpu.VMEM((2, C, 16), jnp.float32),
        pltpu.VMEM((2, C, 16), jnp.float32),
        pltpu.VMEM((2, C, D), jnp.float32),
        pltpu.VMEM((C, D), jnp.float32),
        pltpu.VMEM((C, 16), jnp.float32),
        pltpu.VMEM_SHARED((NP, D), jnp.float32),
        pltpu.VMEM_SHARED((NP, 16), jnp.float32),
        pltpu.SemaphoreType.DMA,
        pltpu.SemaphoreType.DMA,
        pltpu.SemaphoreType.DMA,
        pltpu.SemaphoreType.DMA,
        ],
    )


def _sc_edge(h, als, ald, srcp, dstp):
    return _get_sc_edge()(h, als, ald, srcp, dstp)


def _s16():
    j = lax.broadcasted_iota(jnp.int32, (D, 16), 0)
    i = lax.broadcasted_iota(jnp.int32, (D, 16), 1)
    return (i == j // OC).astype(jnp.float32)


def _rexp():
    i = lax.broadcasted_iota(jnp.int32, (16, D), 0)
    j = lax.broadcasted_iota(jnp.int32, (16, D), 1)
    return (i == j // OC).astype(jnp.float32)


def _mhead():
    j = lax.broadcasted_iota(jnp.int32, (D, 16), 0)
    i = lax.broadcasted_iota(jnp.int32, (D, 16), 1)
    return (i == j % OC).astype(jnp.float32) / H


def _elu(x):
    return jnp.where(x > 0, x, jnp.exp(x) - 1.0)


def _attn_logits(h, asv, adv):
    s16 = _s16()
    als = jnp.dot(h * asv, s16, preferred_element_type=jnp.float32)
    ald = jnp.dot(h * adv, s16, preferred_element_type=jnp.float32)
    return als, ald


def _p0_body(x_ref, w_ref, asv_ref, adv_ref, h_ref, as_ref, ad_ref):
    h = jnp.dot(x_ref[...], w_ref[...], preferred_element_type=jnp.float32)
    h_ref[...] = h
    als, ald = _attn_logits(h, asv_ref[...], adv_ref[...])
    as_ref[...] = als
    ad_ref[...] = ald


_p0 = pl.pallas_call(
    _p0_body,
    grid=(NB,),
    in_specs=[
        pl.BlockSpec((BN, D), lambda b: (b, 0)),
        pl.BlockSpec((D, D), lambda b: (0, 0)),
        pl.BlockSpec((1, D), lambda b: (0, 0)),
        pl.BlockSpec((1, D), lambda b: (0, 0)),
    ],
    out_specs=[
        pl.BlockSpec((BN, D), lambda b: (b, 0)),
        pl.BlockSpec((BN, 16), lambda b: (b, 0)),
        pl.BlockSpec((BN, 16), lambda b: (b, 0)),
    ],
    out_shape=[
        jax.ShapeDtypeStruct((NP, D), jnp.float32),
        jax.ShapeDtypeStruct((NP, 16), jnp.float32),
        jax.ShapeDtypeStruct((NP, 16), jnp.float32),
    ],
)


def _f_body(accm_ref, accd_ref, bias_ref, g_ref, be_ref,
            wn_ref, asv_ref, adv_ref,
            h_ref, as_ref, ad_ref,
            stash, s1, s2):
    p = pl.program_id(0)
    b = pl.program_id(1)

    @pl.when(p == 0)
    def _():
        @pl.when(b == 0)
        def _():
            s1[...] = jnp.zeros_like(s1)
            s2[...] = jnp.zeros_like(s2)
        a = accm_ref[0] + accm_ref[1]
        den = accd_ref[0] + accd_ref[1]
        den_rep = jnp.dot(den, _rexp(), preferred_element_type=jnp.float32)
        out = a / (den_rep + 1e-16) + bias_ref[...]
        rows = lax.broadcasted_iota(jnp.int32, (BN, 1), 0) + b * BN
        out = jnp.where(rows < N, out, 0.0)
        stash[pl.ds(b * BN, BN), :] = out
        s1[...] += jnp.sum(out, axis=0, keepdims=True)
        s2[...] += jnp.sum(out * out, axis=0, keepdims=True)

    @pl.when(p == 1)
    def _():
        mu = s1[...] / N
        var = s2[...] / N - mu * mu
        inv = g_ref[...] / jnp.sqrt(var + 1e-5)
        y = (stash[pl.ds(b * BN, BN), :] - mu) * inv + be_ref[...]
        y = _elu(y)
        hn = jnp.dot(y, wn_ref[...], preferred_element_type=jnp.float32)
        h_ref[...] = hn
        als, ald = _attn_logits(hn, asv_ref[...], adv_ref[...])
        as_ref[...] = als
        ad_ref[...] = ald


_f_layer = pl.pallas_call(
    _f_body,
    grid=(2, NB),
    in_specs=[
        pl.BlockSpec((2, BN, D), lambda p, b: (0, b, 0)),
        pl.BlockSpec((2, BN, 16), lambda p, b: (0, b, 0)),
        pl.BlockSpec((1, D), lambda p, b: (0, 0)),
        pl.BlockSpec((1, D), lambda p, b: (0, 0)),
        pl.BlockSpec((1, D), lambda p, b: (0, 0)),
        pl.BlockSpec((D, D), lambda p, b: (0, 0)),
        pl.BlockSpec((1, D), lambda p, b: (0, 0)),
        pl.BlockSpec((1, D), lambda p, b: (0, 0)),
    ],
    out_specs=[
        pl.BlockSpec((BN, D), lambda p, b: (b, 0)),
        pl.BlockSpec((BN, 16), lambda p, b: (b, 0)),
        pl.BlockSpec((BN, 16), lambda p, b: (b, 0)),
    ],
    out_shape=[
        jax.ShapeDtypeStruct((NP, D), jnp.float32),
        jax.ShapeDtypeStruct((NP, 16), jnp.float32),
        jax.ShapeDtypeStruct((NP, 16), jnp.float32),
    ],
    scratch_shapes=[
        pltpu.VMEM((NP, D), jnp.float32),
        pltpu.VMEM((1, D), jnp.float32),
        pltpu.VMEM((1, D), jnp.float32),
    ],
)


def _f3_body(accm_ref, accd_ref, bias_ref, g_ref, be_ref, batch_ref,
             fc1w_ref, fc1b_ref, fc2w_ref, fc2b_ref,
             out_ref,
             stash, s1, s2, pooled, cnt):
    p = pl.program_id(0)
    b = pl.program_id(1)

    @pl.when(p == 0)
    def _():
        @pl.when(b == 0)
        def _():
            s1[...] = jnp.zeros_like(s1)
            s2[...] = jnp.zeros_like(s2)
        a = accm_ref[0] + accm_ref[1]
        den = accd_ref[0] + accd_ref[1]
        den_rep = jnp.dot(den, _rexp(), preferred_element_type=jnp.float32)
        out128 = a / (den_rep + 1e-16)
        out16 = jnp.dot(out128, _mhead(), preferred_element_type=jnp.float32)
        out16 = out16 + bias_ref[...]
        rows = lax.broadcasted_iota(jnp.int32, (BN, 1), 0) + b * BN
        out16 = jnp.where(rows < N, out16, 0.0)
        stash[pl.ds(b * BN, BN), :] = out16
        s1[...] += jnp.sum(out16, axis=0, keepdims=True)
        s2[...] += jnp.sum(out16 * out16, axis=0, keepdims=True)

    @pl.when(p == 1)
    def _():
        @pl.when(b == 0)
        def _():
            pooled[...] = jnp.zeros_like(pooled)
            cnt[...] = jnp.zeros_like(cnt)
        mu = s1[...] / N
        var = s2[...] / N - mu * mu
        inv = g_ref[...] / jnp.sqrt(var + 1e-5)
        y = (stash[pl.ds(b * BN, BN), :] - mu) * inv + be_ref[...]
        y = _elu(y)
        seg = batch_ref[...]
        gcol = lax.broadcasted_iota(jnp.int32, (BN, G), 1)
        onehot = (seg == gcol).astype(jnp.float32)
        pooled[...] += lax.dot_general(
            onehot, y, (((0,), (0,)), ((), ())),
            preferred_element_type=jnp.float32)
        cnt[...] += jnp.sum(onehot, axis=0, keepdims=True)

        @pl.when(b == NB - 1)
        def _():
            cnts = jnp.maximum(cnt[...], 1.0)
            pm = pooled[...] / cnts.reshape(G, 1)
            o = _elu(jnp.dot(pm, fc1w_ref[...],
                             preferred_element_type=jnp.float32)
                     + fc1b_ref[...])
            out_ref[...] = (jnp.dot(o, fc2w_ref[...],
                                    preferred_element_type=jnp.float32)
                            + fc2b_ref[...])


_f3 = pl.pallas_call(
    _f3_body,
    grid=(2, NB),
    in_specs=[
        pl.BlockSpec((2, BN, D), lambda p, b: (0, b, 0)),
        pl.BlockSpec((2, BN, 16), lambda p, b: (0, b, 0)),
        pl.BlockSpec((1, 16), lambda p, b: (0, 0)),
        pl.BlockSpec((1, 16), lambda p, b: (0, 0)),
        pl.BlockSpec((1, 16), lambda p, b: (0, 0)),
        pl.BlockSpec((BN, 1), lambda p, b: (b, 0)),
        pl.BlockSpec((OC, D), lambda p, b: (0, 0)),
        pl.BlockSpec((1, D), lambda p, b: (0, 0)),
        pl.BlockSpec((D, D), lambda p, b: (0, 0)),
        pl.BlockSpec((1, D), lambda p, b: (0, 0)),
    ],
    out_specs=pl.BlockSpec((G, D), lambda p, b: (0, 0)),
    out_shape=jax.ShapeDtypeStruct((G, D), jnp.float32),
    scratch_shapes=[
        pltpu.VMEM((NP, 16), jnp.float32),
        pltpu.VMEM((1, 16), jnp.float32),
        pltpu.VMEM((1, 16), jnp.float32),
        pltpu.VMEM((G, 16), jnp.float32),
        pltpu.VMEM((1, G), jnp.float32),
    ],
)


def kernel(x, W0, as0, ad0, b0, W1, as1, ad1, b1, W2, as2, ad2, b2,
           W3, as3, ad3, b3, g0, be0, g1, be1, g2, be2, g3, be3,
           fc1_w, fc1_b, fc2_w, fc2_b, edge_index, batch):
    f32 = jnp.float32
    x_pad = jnp.zeros((NP, D), f32).at[:N].set(x.astype(f32))
    src = edge_index[0].astype(jnp.int32)
    dst = edge_index[1].astype(jnp.int32)
    pad = jnp.full((EP - E,), DUMMY, jnp.int32)
    dchunk = jnp.full((NW, 2, C), DUMMY, jnp.int32)
    srcp = jnp.concatenate(
        [jnp.concatenate([src, pad]).reshape(NW, NCH, C), dchunk], axis=1)
    dstp = jnp.concatenate(
        [jnp.concatenate([dst, pad]).reshape(NW, NCH, C), dchunk], axis=1)
    batch_pad = jnp.full((NP, 1), 999, jnp.int32).at[:N, 0].set(
        batch.astype(jnp.int32))

    Ws = [W0, W1, W2, W3]
    asvs = [a.reshape(1, D) for a in (as0, as1, as2, as3)]
    advs = [a.reshape(1, D) for a in (ad0, ad1, ad2, ad3)]
    biases = [b0.reshape(1, D), b1.reshape(1, D), b2.reshape(1, D)]
    gs = [g0.reshape(1, D), g1.reshape(1, D), g2.reshape(1, D)]
    bes = [be0.reshape(1, D), be1.reshape(1, D), be2.reshape(1, D)]

    h, als, ald = _p0(x_pad, Ws[0], asvs[0], advs[0])
    for i in range(3):
        accm, accd = _sc_edge(h, als, ald, srcp, dstp)
        h, als, ald = _f_layer(accm, accd, biases[i], gs[i], bes[i],
                               Ws[i + 1], asvs[i + 1], advs[i + 1])
    accm, accd = _sc_edge(h, als, ald, srcp, dstp)
    out = _f3(accm, accd, b3.reshape(1, OC), g3.reshape(1, OC),
              be3.reshape(1, OC), batch_pad,
              fc1_w, fc1_b.reshape(1, D), fc2_w, fc2_b.reshape(1, D))
    return out
